# Initial kernel scaffold; baseline (speedup 1.0000x reference)
import jax
import jax.numpy as jnp
from jax.experimental import pallas as pl


def kernel(x, edge_index, W_scalar_rel, W_scalar_root, b_scalar_root, W_vector_rel, W_vector_root):
    raise NotImplementedError("write your pallas kernel here")


# speedup vs baseline: 1.2784x; 1.2784x over previous
import jax
import jax.numpy as jnp
from jax import lax
from jax.experimental import pallas as pl
from jax.experimental.pallas import tpu as pltpu
from jax.experimental.pallas import tpu_sc as plsc

N = 10000
E = 160000
H = 128
D = 4 * H

NC = 2
NS = 16
NW = NC * NS

PASSES = 2
OWN = 168
ACC_R = 176
DUMP = 168
NPAD = PASSES * NW * OWN

STRIPE = 4000
NSTRIPES = E // STRIPE
SCAN_STEPS = STRIPE // 16

G = 16
LROWS = 32


def _sc_body(feat_hbm, rows_hbm, cols_hbm, a_hbm,
             rows_v, cols_v, lidx_v, gcol_v, stage_v, acc_v, sem_a, sem_b):
  c = lax.axis_index("c")
  s = lax.axis_index("s")
  w = c * NS + s

  iota16 = lax.broadcasted_iota(jnp.int32, (16,), 0)
  zvec = jnp.zeros((16,), jnp.float32)

  def _pass(p, carry0):
    base = (p * NW + w) * OWN

    def _zr(i, carry):
      for k in range(D // 16):
        acc_v[i, pl.ds(k * 16, 16)] = zvec
      return carry

    lax.fori_loop(0, ACC_R, _zr, 0)

    def _stripe(st, carry):
      off = st * STRIPE
      pltpu.sync_copy(rows_hbm.at[pl.ds(off, STRIPE)], rows_v)
      pltpu.sync_copy(cols_hbm.at[pl.ds(off, STRIPE)], cols_v)

      def _scan(i, cnt):
        rv = rows_v[pl.ds(i * 16, 16)]
        lr = rv - base
        m = (lr >= 0) & (lr < OWN)
        pc = plsc.all_reduce_population_count(m)[0]

        @pl.when(pc > 0)
        def _():
          cv = cols_v[pl.ds(i * 16, 16)]
          inc = plsc.cumsum(jnp.where(m, 1, 0).astype(jnp.int32))
          tgt = cnt + inc - 1
          plsc.store_scatter(lidx_v, [tgt >> 7, tgt & 127], lr, mask=m)
          plsc.store_scatter(gcol_v, [tgt >> 7, tgt & 127], cv, mask=m)

        return cnt + pc

      cnt = lax.fori_loop(0, SCAN_STEPS, _scan, jnp.int32(0))

      cnt_pad = ((cnt + (G - 1)) // G) * G
      tgt = cnt + iota16
      m = tgt < cnt_pad
      plsc.store_scatter(lidx_v, [tgt >> 7, tgt & 127],
                         jnp.zeros((16,), jnp.int32) + DUMP, mask=m)
      plsc.store_scatter(gcol_v, [tgt >> 7, tgt & 127],
                         jnp.zeros((16,), jnp.int32), mask=m)
      nch = cnt_pad >> 4

      def _gidx(g):
        return gcol_v.at[g >> 3, pl.ds(pl.multiple_of((g & 7) * 16, 16), 16)]

      @pl.when(nch > 0)
      def _():
        pltpu.async_copy(feat_hbm.at[_gidx(0)], stage_v.at[0], sem_a)

      def _chunk(g, carry):
        pb = g & 1

        @pl.when(pb == 0)
        def _():
          pltpu.make_async_copy(feat_hbm.at[_gidx(g)], stage_v.at[0],
                                sem_a).wait()

        @pl.when(pb == 1)
        def _():
          pltpu.make_async_copy(feat_hbm.at[_gidx(g)], stage_v.at[1],
                                sem_b).wait()

        @pl.when((g + 1 < nch) & (pb == 0))
        def _():
          pltpu.async_copy(feat_hbm.at[_gidx(g + 1)], stage_v.at[1], sem_b)

        @pl.when((g + 1 < nch) & (pb == 1))
        def _():
          pltpu.async_copy(feat_hbm.at[_gidx(g + 1)], stage_v.at[0], sem_a)

        lv = lidx_v[g >> 3, pl.ds(pl.multiple_of((g & 7) * 16, 16), 16)]
        for j in range(G):
          r = lv[j]
          for k in range(D // 16):
            plsc.addupdate(acc_v.at[r, pl.ds(k * 16, 16)],
                           stage_v[pb, j, pl.ds(k * 16, 16)])
        return carry

      lax.fori_loop(0, nch, _chunk, 0)
      return carry

    lax.fori_loop(0, NSTRIPES, _stripe, 0)

    pltpu.sync_copy(acc_v.at[pl.ds(0, OWN)],
                    a_hbm.at[pl.ds(pl.multiple_of(base, 8), OWN)])
    return carry0

  lax.fori_loop(0, PASSES, _pass, 0)


@jax.jit
def _sc_aggregate(feat, rows, cols):
  mesh = plsc.VectorSubcoreMesh(core_axis_name="c", subcore_axis_name="s")
  return pl.kernel(
      _sc_body,
      out_type=jax.ShapeDtypeStruct((NPAD, D), jnp.float32),
      mesh=mesh,
      scratch_types=[
          pltpu.VMEM((STRIPE,), jnp.int32),
          pltpu.VMEM((STRIPE,), jnp.int32),
          pltpu.VMEM((LROWS, 128), jnp.int32),
          pltpu.VMEM((LROWS, 128), jnp.int32),
          pltpu.VMEM((2, G, D), jnp.float32),
          pltpu.VMEM((ACC_R, D), jnp.float32),
          pltpu.SemaphoreType.DMA,
          pltpu.SemaphoreType.DMA,
      ],
      compiler_params=pltpu.CompilerParams(needs_layout_passes=False),
  )(feat, rows, cols)


BLK = 400


def _tc_body(feat_ref, a_ref, wsr_ref, wsrel_ref, b_ref, wvr_ref, wvrel_ref,
             out_ref):
  f = feat_ref[...]
  a = a_ref[...]
  s_out = (jnp.dot(f[:, :H], wsr_ref[...], preferred_element_type=jnp.float32)
           + jnp.dot(a[:, :H], wsrel_ref[...],
                     preferred_element_type=jnp.float32)
           + b_ref[...])
  v_out = (jnp.dot(f[:, H:], wvr_ref[...], preferred_element_type=jnp.float32)
           + jnp.dot(a[:, H:], wvrel_ref[...],
                     preferred_element_type=jnp.float32))
  out_ref[:, :H] = s_out
  out_ref[:, H:] = v_out


@jax.jit
def _tc_linear(feat, a_pad, wsr, wsrel, b, wvr, wvrel):
  return pl.pallas_call(
      _tc_body,
      grid=(N // BLK,),
      in_specs=[
          pl.BlockSpec((BLK, D), lambda i: (i, 0)),
          pl.BlockSpec((BLK, D), lambda i: (i, 0)),
          pl.BlockSpec((H, H), lambda i: (0, 0)),
          pl.BlockSpec((H, H), lambda i: (0, 0)),
          pl.BlockSpec((1, H), lambda i: (0, 0)),
          pl.BlockSpec((3 * H, 3 * H), lambda i: (0, 0)),
          pl.BlockSpec((3 * H, 3 * H), lambda i: (0, 0)),
      ],
      out_specs=pl.BlockSpec((BLK, D), lambda i: (i, 0)),
      out_shape=jax.ShapeDtypeStruct((N, D), jnp.float32),
  )(feat, a_pad, wsr, wsrel, b, wvr, wvrel)


def kernel(x, edge_index, W_scalar_rel, W_scalar_root, b_scalar_root,
           W_vector_rel, W_vector_root):
  feat = x.reshape(N, D)
  ei = edge_index.astype(jnp.int32)
  rows = ei[0]
  cols = ei[1]
  agg = _sc_aggregate(feat, rows, cols)
  out = _tc_linear(feat, agg, W_scalar_root, W_scalar_rel,
                   b_scalar_root.reshape(1, H), W_vector_root, W_vector_rel)
  return out.reshape(N, 4, H)


# speedup vs baseline: 1.4432x; 1.1289x over previous
import jax
import jax.numpy as jnp
from jax import lax
from jax.experimental import pallas as pl
from jax.experimental.pallas import tpu as pltpu
from jax.experimental.pallas import tpu_sc as plsc

N = 10000
E = 160000
H = 128
D = 4 * H

NC = 2
NS = 16
NW = NC * NS

PASSES = 2
OWN = 168
ACC_R = 176
DUMP = 168
NPAD = PASSES * NW * OWN

STRIPE = 3200
NSTRIPES = E // STRIPE
SCAN_STEPS = STRIPE // 64

G = 16
LROWS = 32


def _sc_body(feat_hbm, rows_hbm, cols_hbm, a_hbm,
             edg_v, lidx_v, gcol_v, stage_v, acc_v,
             sem_a, sem_b, sem_e0, sem_e1):
  c = lax.axis_index("c")
  s = lax.axis_index("s")
  w = c * NS + s

  iota16 = lax.broadcasted_iota(jnp.int32, (16,), 0)
  zvec = jnp.zeros((16,), jnp.float32)

  def _edge_refs(st, eb):
    src_r = rows_hbm.at[pl.ds(st * STRIPE, STRIPE)]
    src_c = cols_hbm.at[pl.ds(st * STRIPE, STRIPE)]
    return (src_r, edg_v.at[eb, 0], src_c, edg_v.at[eb, 1])

  def _fire_edges(st, eb):
    sr, dr, sc_, dc = _edge_refs(st, eb)

    @pl.when(eb == 0)
    def _():
      pltpu.async_copy(sr, dr, sem_e0)
      pltpu.async_copy(sc_, dc, sem_e0)

    @pl.when(eb == 1)
    def _():
      pltpu.async_copy(sr, dr, sem_e1)
      pltpu.async_copy(sc_, dc, sem_e1)

  def _wait_edges(st, eb):
    sr, dr, sc_, dc = _edge_refs(st, eb)

    @pl.when(eb == 0)
    def _():
      pltpu.make_async_copy(sr, dr, sem_e0).wait()
      pltpu.make_async_copy(sc_, dc, sem_e0).wait()

    @pl.when(eb == 1)
    def _():
      pltpu.make_async_copy(sr, dr, sem_e1).wait()
      pltpu.make_async_copy(sc_, dc, sem_e1).wait()

  def _gidx(g):
    return gcol_v.at[g >> 3, pl.ds(pl.multiple_of((g & 7) * 16, 16), 16)]

  def _lidx(g):
    return lidx_v[g >> 3, pl.ds(pl.multiple_of((g & 7) * 16, 16), 16)]

  def _run_chunks(nch):
    @pl.when(nch > 0)
    def _():
      pltpu.async_copy(feat_hbm.at[_gidx(0)], stage_v.at[0], sem_a)

    def _chunk(g, carry):
      pb = g & 1

      @pl.when(pb == 0)
      def _():
        pltpu.make_async_copy(feat_hbm.at[_gidx(g)], stage_v.at[0],
                              sem_a).wait()

      @pl.when(pb == 1)
      def _():
        pltpu.make_async_copy(feat_hbm.at[_gidx(g)], stage_v.at[1],
                              sem_b).wait()

      @pl.when((g + 1 < nch) & (pb == 0))
      def _():
        pltpu.async_copy(feat_hbm.at[_gidx(g + 1)], stage_v.at[1], sem_b)

      @pl.when((g + 1 < nch) & (pb == 1))
      def _():
        pltpu.async_copy(feat_hbm.at[_gidx(g + 1)], stage_v.at[0], sem_a)

      lv = _lidx(g)
      for j in range(G):
        r = lv[j]
        for k in range(D // 16):
          plsc.addupdate(acc_v.at[r, pl.ds(k * 16, 16)],
                         stage_v[pb, j, pl.ds(k * 16, 16)])
      return carry

    lax.fori_loop(0, nch, _chunk, 0)

  def _pass(p, carry0):
    base = (p * NW + w) * OWN

    def _zr(i, carry):
      for k in range(D // 16):
        acc_v[i, pl.ds(k * 16, 16)] = zvec
      return carry

    lax.fori_loop(0, ACC_R, _zr, 0)

    _fire_edges(0, 0)

    def _stripe(st, cnt):
      eb = st & 1
      _wait_edges(st, eb)

      @pl.when(st + 1 < NSTRIPES)
      def _():
        _fire_edges(st + 1, 1 - eb)

      def _scan(i, tot):
        for k in range(4):
          o = i * 64 + k * 16
          rv = edg_v[eb, 0, pl.ds(o, 16)]
          lr = rv - base
          m = (lr >= 0) & (lr < OWN)
          pc = plsc.all_reduce_population_count(m)
          tot_k = tot

          @pl.when(pc[0] > 0)
          def _():
            cv = edg_v[eb, 1, pl.ds(o, 16)]
            inc = plsc.cumsum(jnp.where(m, 1, 0).astype(jnp.int32))
            tgt = tot_k + inc - 1
            plsc.store_scatter(lidx_v, [tgt >> 7, tgt & 127], lr, mask=m)
            plsc.store_scatter(gcol_v, [tgt >> 7, tgt & 127], cv, mask=m)

          tot = tot + pc
        return tot

      cnt2 = lax.fori_loop(0, SCAN_STEPS, _scan, cnt)

      nch = cnt2[0] >> 4
      _run_chunks(nch)
      rem = cnt2 - nch * 16
      lv_l = _lidx(nch)
      gv_l = gcol_v[nch >> 3, pl.ds(pl.multiple_of((nch & 7) * 16, 16), 16)]
      mrem = iota16 < rem
      plsc.store_scatter(lidx_v, [iota16 * 0, iota16], lv_l, mask=mrem)
      plsc.store_scatter(gcol_v, [iota16 * 0, iota16], gv_l, mask=mrem)
      return rem

    cnt = lax.fori_loop(0, NSTRIPES, _stripe,
                        jnp.zeros((16,), jnp.int32))

    tgt = cnt + iota16
    cnt_pad = ((cnt[0] + (G - 1)) // G) * G
    m = tgt < cnt_pad
    plsc.store_scatter(lidx_v, [tgt >> 7, tgt & 127],
                       jnp.zeros((16,), jnp.int32) + DUMP, mask=m)
    plsc.store_scatter(gcol_v, [tgt >> 7, tgt & 127],
                       jnp.zeros((16,), jnp.int32), mask=m)
    _run_chunks(cnt_pad >> 4)

    pltpu.sync_copy(acc_v.at[pl.ds(0, OWN)],
                    a_hbm.at[pl.ds(pl.multiple_of(base, 8), OWN)])
    return carry0

  lax.fori_loop(0, PASSES, _pass, 0)


@jax.jit
def _sc_aggregate(feat, rows, cols):
  mesh = plsc.VectorSubcoreMesh(core_axis_name="c", subcore_axis_name="s")
  return pl.kernel(
      _sc_body,
      out_type=jax.ShapeDtypeStruct((NPAD, D), jnp.float32),
      mesh=mesh,
      scratch_types=[
          pltpu.VMEM((2, 2, STRIPE), jnp.int32),
          pltpu.VMEM((LROWS, 128), jnp.int32),
          pltpu.VMEM((LROWS, 128), jnp.int32),
          pltpu.VMEM((2, G, D), jnp.float32),
          pltpu.VMEM((ACC_R, D), jnp.float32),
          pltpu.SemaphoreType.DMA,
          pltpu.SemaphoreType.DMA,
          pltpu.SemaphoreType.DMA,
          pltpu.SemaphoreType.DMA,
      ],
      compiler_params=pltpu.CompilerParams(needs_layout_passes=False),
  )(feat, rows, cols)


BLK = 400


def _tc_body(feat_ref, a_ref, wsr_ref, wsrel_ref, b_ref, wvr_ref, wvrel_ref,
             out_ref):
  f = feat_ref[...]
  a = a_ref[...]
  s_out = (jnp.dot(f[:, :H], wsr_ref[...], preferred_element_type=jnp.float32)
           + jnp.dot(a[:, :H], wsrel_ref[...],
                     preferred_element_type=jnp.float32)
           + b_ref[...])
  v_out = (jnp.dot(f[:, H:], wvr_ref[...], preferred_element_type=jnp.float32)
           + jnp.dot(a[:, H:], wvrel_ref[...],
                     preferred_element_type=jnp.float32))
  out_ref[:, :H] = s_out
  out_ref[:, H:] = v_out


@jax.jit
def _tc_linear(feat, a_pad, wsr, wsrel, b, wvr, wvrel):
  return pl.pallas_call(
      _tc_body,
      grid=(N // BLK,),
      in_specs=[
          pl.BlockSpec((BLK, D), lambda i: (i, 0)),
          pl.BlockSpec((BLK, D), lambda i: (i, 0)),
          pl.BlockSpec((H, H), lambda i: (0, 0)),
          pl.BlockSpec((H, H), lambda i: (0, 0)),
          pl.BlockSpec((1, H), lambda i: (0, 0)),
          pl.BlockSpec((3 * H, 3 * H), lambda i: (0, 0)),
          pl.BlockSpec((3 * H, 3 * H), lambda i: (0, 0)),
      ],
      out_specs=pl.BlockSpec((BLK, D), lambda i: (i, 0)),
      out_shape=jax.ShapeDtypeStruct((N, D), jnp.float32),
  )(feat, a_pad, wsr, wsrel, b, wvr, wvrel)


def kernel(x, edge_index, W_scalar_rel, W_scalar_root, b_scalar_root,
           W_vector_rel, W_vector_root):
  feat = x.reshape(N, D)
  ei = edge_index.astype(jnp.int32)
  rows = ei[0]
  cols = ei[1]
  agg = _sc_aggregate(feat, rows, cols)
  out = _tc_linear(feat, agg, W_scalar_root, W_scalar_rel,
                   b_scalar_root.reshape(1, H), W_vector_root, W_vector_rel)
  return out.reshape(N, 4, H)
